# baseline (speedup 1.0000x reference)
import functools

import jax
import jax.numpy as jnp
from jax import lax
from jax.experimental import pallas as pl
from jax.experimental.pallas import tpu as pltpu
from jax.experimental.pallas import tpu_sc as plsc

N = 50000
K = 64
DE = 16
DN = 128

MP = 400
NP = MP * 128

NC = 2
NS = 16
NW = NC * NS
CPT = K // NW
CROWS = 40
NCHUNK = MP // CROWS

NB_A = 4096
MB_A = NB_A // 128
NB_C = 4096
MB_C = NB_C // 128

_I32 = jnp.int32
_F32 = jnp.float32
_BF16 = jnp.bfloat16
_HI = -65536


def _stage_a_body(eh_ref, mask_ref, x_ref, idx_ref, wmat_ref, bu_ref,
                  wT_ref, xi_ref):
    dX = jnp.dot(eh_ref[...], wmat_ref[...],
                 preferred_element_type=_F32)
    z = dX + bu_ref[0, 0]
    w = (1.0 / (1.0 + jnp.exp(-z))) * mask_ref[...]
    wn = w / (jnp.sum(w, axis=-1, keepdims=True) + 1e-6)
    wT_ref[...] = wn.T.astype(_BF16).reshape(K, MB_A, 128)

    xt = x_ref[...].T.reshape(NW, CPT, NB_A)
    xe = lax.bitcast_convert_type(xt[:, 0, :], _I32)
    xo = lax.bitcast_convert_type(xt[:, 1, :], _I32)
    xpk = lax.shift_right_logical(xe, 16) | (xo & _HI)

    it = idx_ref[...].T.reshape(NW, CPT, NB_A)
    ipk = (it[:, 0, :] & 0xFFFF) | (it[:, 1, :] << 16)
    xi_ref[...] = jnp.stack([xpk, ipk]).reshape(2, NW, MB_A, 128)


def _stage_a(eh2, mask2, x2, idx2, wmat, bu):
    grid = (pl.cdiv(N, NB_A),)
    return pl.pallas_call(
        _stage_a_body,
        grid=grid,
        in_specs=[
            pl.BlockSpec((NB_A, K * DE), lambda i: (i, 0)),
            pl.BlockSpec((NB_A, K), lambda i: (i, 0)),
            pl.BlockSpec((NB_A, K), lambda i: (i, 0)),
            pl.BlockSpec((NB_A, K), lambda i: (i, 0)),
            pl.BlockSpec((K * DE, K), lambda i: (0, 0)),
            pl.BlockSpec(memory_space=pltpu.SMEM),
        ],
        out_specs=[
            pl.BlockSpec((K, MB_A, 128), lambda i: (0, i, 0)),
            pl.BlockSpec((2, NW, MB_A, 128), lambda i: (0, 0, i, 0)),
        ],
        out_shape=[
            jax.ShapeDtypeStruct((K, MP, 128), _BF16),
            jax.ShapeDtypeStruct((2, NW, MP, 128), _I32),
        ],
    )(eh2, mask2, x2, idx2, wmat, bu)


def _sc_body(xi, out, xcols, idxb, resb, isem0, isem1, osem0, osem1):
    cid = lax.axis_index("c")
    sid = lax.axis_index("s")
    wid = sid * NC + cid
    isems = (isem0, isem1)
    osems = (osem0, osem1)

    pltpu.sync_copy(xi.at[0, wid], xcols)

    def issue_load(j, b):
        pltpu.async_copy(xi.at[1, wid, pl.ds(j * CROWS, CROWS)],
                         idxb.at[b], isems[b])

    def wait_load(b):
        pltpu.make_async_copy(xi.at[1, wid, pl.ds(0, CROWS)],
                              idxb.at[b], isems[b]).wait()

    def out_window(j):
        return out.at[wid, pl.ds(j * CROWS, CROWS)]

    issue_load(0, 0)

    def super_body(jj, carry):
        for b in range(2):
            j = 2 * jj + b

            @pl.when(j + 1 < NCHUNK)
            def _():
                issue_load(j + 1, 1 - b)

            @pl.when(j >= 2)
            def _():
                pltpu.make_async_copy(resb.at[b], out_window(j - 2),
                                      osems[b]).wait()

            wait_load(b)

            def vec_body(i, c2):
                r = i >> 3
                l = (i & 7) * 16
                s = pl.ds(l, 16)
                pv = idxb[b, r, s]
                g0 = jnp.minimum(pv & 0xFFFF, N - 1)
                g1 = jnp.minimum(lax.shift_right_logical(pv, 16), N - 1)
                p0 = plsc.load_gather(
                    xcols, [lax.shift_right_logical(g0, 7),
                            lax.bitwise_and(g0, 127)])
                p1 = plsc.load_gather(
                    xcols, [lax.shift_right_logical(g1, 7),
                            lax.bitwise_and(g1, 127)])
                resb[b, r, s] = (p0 & 0xFFFF) | (p1 & _HI)
                return c2

            lax.fori_loop(0, CROWS * 8, vec_body, 0)
            pltpu.async_copy(resb.at[b], out_window(j), osems[b])
        return carry

    lax.fori_loop(0, NCHUNK // 2, super_body, 0)
    pltpu.make_async_copy(resb.at[0], out_window(NCHUNK - 2), osems[0]).wait()
    pltpu.make_async_copy(resb.at[1], out_window(NCHUNK - 1), osems[1]).wait()


@functools.cache
def _sc_gather():
    return pl.kernel(
        _sc_body,
        out_type=jax.ShapeDtypeStruct((NW, MP, 128), _I32),
        mesh=plsc.VectorSubcoreMesh(core_axis_name="c", subcore_axis_name="s",
                                    num_cores=NC, num_subcores=NS),
        scratch_types=[
            pltpu.VMEM((MP, 128), _I32),
            pltpu.VMEM((2, CROWS, 128), _I32),
            pltpu.VMEM((2, CROWS, 128), _I32),
            pltpu.SemaphoreType.DMA,
            pltpu.SemaphoreType.DMA,
            pltpu.SemaphoreType.DMA,
            pltpu.SemaphoreType.DMA,
        ],
        compiler_params=pltpu.CompilerParams(use_tc_tiling_on_sc=False,
                                             needs_layout_passes=False),
    )


def _stage_c_body(wT_ref, xn_ref, nh_ref, wr_ref, br_ref, out_ref):
    w3 = wT_ref[...].astype(_F32).reshape(NW, CPT, NB_C)
    xn = xn_ref[...].reshape(NW, NB_C)
    x0 = lax.bitcast_convert_type(xn << 16, _F32)
    x1 = lax.bitcast_convert_type(xn & _HI, _F32)
    contrib = w3[:, 0, :] * x0 + w3[:, 1, :] * x1
    ps = jnp.sum(contrib, axis=0)
    refine = jnp.sum(nh_ref[...] * wr_ref[...], axis=-1)
    out_ref[...] = (ps + refine + br_ref[0, 0])[None, :]


def _stage_c(wT, xn, nh2, wr, br):
    grid = (pl.cdiv(N, NB_C),)
    return pl.pallas_call(
        _stage_c_body,
        grid=grid,
        in_specs=[
            pl.BlockSpec((K, MB_C, 128), lambda i: (0, i, 0)),
            pl.BlockSpec((NW, MB_C, 128), lambda i: (0, i, 0)),
            pl.BlockSpec((NB_C, DN), lambda i: (i, 0)),
            pl.BlockSpec((1, DN), lambda i: (0, 0)),
            pl.BlockSpec(memory_space=pltpu.SMEM),
        ],
        out_specs=pl.BlockSpec((1, NB_C), lambda i: (0, i)),
        out_shape=jax.ShapeDtypeStruct((1, N), _F32),
    )(wT, xn, nh2, wr, br)


def kernel(X, node_h, edge_h, edge_idx, mask_i, mask_ij, Wu_w, Wu_b, Wr_w, Wr_b):
    eh2 = edge_h.reshape(N, K * DE)
    mask2 = mask_ij.reshape(N, K)
    x2 = X.reshape(N, K)
    idx2 = edge_idx.reshape(N, K).astype(_I32)
    nh2 = node_h.reshape(N, DN)
    wmat = jnp.kron(jnp.eye(K, dtype=_F32), Wu_w.reshape(DE, 1))
    bu = Wu_b.reshape(1, 1)
    br = Wr_b.reshape(1, 1)

    wT, xi = _stage_a(eh2, mask2, x2, idx2, wmat, bu)
    xn = _sc_gather()(xi)
    out2 = _stage_c(wT, xn, nh2, Wr_w.reshape(1, DN), br)
    return out2.reshape(1, N, 1)


# speedup vs baseline: 1.0017x; 1.0017x over previous
import functools

import jax
import jax.numpy as jnp
from jax import lax
from jax.experimental import pallas as pl
from jax.experimental.pallas import tpu as pltpu
from jax.experimental.pallas import tpu_sc as plsc

N = 50000
K = 64
DE = 16
DN = 128

MP = 400
NP = MP * 128

NC = 2
NS = 16
NW = NC * NS
CPT = K // NW
CROWS = 50
NCHUNK = MP // CROWS

NB_A = 4096
MB_A = NB_A // 128
NB_C = 4096
MB_C = NB_C // 128

_I32 = jnp.int32
_F32 = jnp.float32
_BF16 = jnp.bfloat16
_HI = -65536


def _stage_a_body(eh_ref, mask_ref, x_ref, idx_ref, wmat_ref, bu_ref,
                  wT_ref, xi_ref):
    dX = jnp.dot(eh_ref[...], wmat_ref[...],
                 preferred_element_type=_F32)
    z = dX + bu_ref[0, 0]
    w = (1.0 / (1.0 + jnp.exp(-z))) * mask_ref[...]
    wn = w / (jnp.sum(w, axis=-1, keepdims=True) + 1e-6)
    wT_ref[...] = wn.T.astype(_BF16).reshape(K, MB_A, 128)

    xt = x_ref[...].T.reshape(NW, CPT, NB_A)
    xe = lax.bitcast_convert_type(xt[:, 0, :], _I32)
    xo = lax.bitcast_convert_type(xt[:, 1, :], _I32)
    xpk = lax.shift_right_logical(xe, 16) | (xo & _HI)

    it = idx_ref[...].T.reshape(NW, CPT, NB_A)
    ipk = (it[:, 0, :] & 0xFFFF) | (it[:, 1, :] << 16)
    xi_ref[...] = jnp.stack([xpk, ipk]).reshape(2, NW, MB_A, 128)


def _stage_a(eh2, mask2, x2, idx2, wmat, bu):
    grid = (pl.cdiv(N, NB_A),)
    return pl.pallas_call(
        _stage_a_body,
        grid=grid,
        in_specs=[
            pl.BlockSpec((NB_A, K * DE), lambda i: (i, 0)),
            pl.BlockSpec((NB_A, K), lambda i: (i, 0)),
            pl.BlockSpec((NB_A, K), lambda i: (i, 0)),
            pl.BlockSpec((NB_A, K), lambda i: (i, 0)),
            pl.BlockSpec((K * DE, K), lambda i: (0, 0)),
            pl.BlockSpec(memory_space=pltpu.SMEM),
        ],
        out_specs=[
            pl.BlockSpec((K, MB_A, 128), lambda i: (0, i, 0)),
            pl.BlockSpec((2, NW, MB_A, 128), lambda i: (0, 0, i, 0)),
        ],
        out_shape=[
            jax.ShapeDtypeStruct((K, MP, 128), _BF16),
            jax.ShapeDtypeStruct((2, NW, MP, 128), _I32),
        ],
    )(eh2, mask2, x2, idx2, wmat, bu)


def _sc_body(xi, out, xcols, idxb, resb, isem0, isem1, osem0, osem1):
    cid = lax.axis_index("c")
    sid = lax.axis_index("s")
    wid = sid * NC + cid
    isems = (isem0, isem1)
    osems = (osem0, osem1)

    def issue_load(j, b):
        pltpu.async_copy(xi.at[1, wid, pl.ds(j * CROWS, CROWS)],
                         idxb.at[b], isems[b])

    def wait_load(b):
        pltpu.make_async_copy(xi.at[1, wid, pl.ds(0, CROWS)],
                              idxb.at[b], isems[b]).wait()

    def out_window(j):
        return out.at[wid, pl.ds(j * CROWS, CROWS)]

    issue_load(0, 0)
    pltpu.sync_copy(xi.at[0, wid], xcols)

    def super_body(jj, carry):
        for b in range(2):
            j = 2 * jj + b

            @pl.when(j + 1 < NCHUNK)
            def _():
                issue_load(j + 1, 1 - b)

            @pl.when(j >= 2)
            def _():
                pltpu.make_async_copy(resb.at[b], out_window(j - 2),
                                      osems[b]).wait()

            wait_load(b)

            def vec_body(i, c2):
                r = i >> 3
                l = (i & 7) * 16
                s = pl.ds(l, 16)
                pv = idxb[b, r, s]
                g0 = jnp.minimum(pv & 0xFFFF, N - 1)
                g1 = jnp.minimum(lax.shift_right_logical(pv, 16), N - 1)
                p0 = plsc.load_gather(
                    xcols, [lax.shift_right_logical(g0, 7),
                            lax.bitwise_and(g0, 127)])
                p1 = plsc.load_gather(
                    xcols, [lax.shift_right_logical(g1, 7),
                            lax.bitwise_and(g1, 127)])
                resb[b, r, s] = (p0 & 0xFFFF) | (p1 & _HI)
                return c2

            lax.fori_loop(0, CROWS * 8, vec_body, 0)
            pltpu.async_copy(resb.at[b], out_window(j), osems[b])
        return carry

    lax.fori_loop(0, NCHUNK // 2, super_body, 0)
    pltpu.make_async_copy(resb.at[0], out_window(NCHUNK - 2), osems[0]).wait()
    pltpu.make_async_copy(resb.at[1], out_window(NCHUNK - 1), osems[1]).wait()


@functools.cache
def _sc_gather():
    return pl.kernel(
        _sc_body,
        out_type=jax.ShapeDtypeStruct((NW, MP, 128), _I32),
        mesh=plsc.VectorSubcoreMesh(core_axis_name="c", subcore_axis_name="s",
                                    num_cores=NC, num_subcores=NS),
        scratch_types=[
            pltpu.VMEM((MP, 128), _I32),
            pltpu.VMEM((2, CROWS, 128), _I32),
            pltpu.VMEM((2, CROWS, 128), _I32),
            pltpu.SemaphoreType.DMA,
            pltpu.SemaphoreType.DMA,
            pltpu.SemaphoreType.DMA,
            pltpu.SemaphoreType.DMA,
        ],
        compiler_params=pltpu.CompilerParams(use_tc_tiling_on_sc=False,
                                             needs_layout_passes=False),
    )


def _stage_c_body(wT_ref, xn_ref, nh_ref, wr_ref, br_ref, out_ref):
    w3 = wT_ref[...].astype(_F32).reshape(NW, CPT, NB_C)
    xn = xn_ref[...].reshape(NW, NB_C)
    x0 = lax.bitcast_convert_type(xn << 16, _F32)
    x1 = lax.bitcast_convert_type(xn & _HI, _F32)
    contrib = w3[:, 0, :] * x0 + w3[:, 1, :] * x1
    ps = jnp.sum(contrib, axis=0)
    refine = jnp.sum(nh_ref[...] * wr_ref[...], axis=-1)
    out_ref[...] = (ps + refine + br_ref[0, 0])[None, :]


def _stage_c(wT, xn, nh2, wr, br):
    grid = (pl.cdiv(N, NB_C),)
    return pl.pallas_call(
        _stage_c_body,
        grid=grid,
        in_specs=[
            pl.BlockSpec((K, MB_C, 128), lambda i: (0, i, 0)),
            pl.BlockSpec((NW, MB_C, 128), lambda i: (0, i, 0)),
            pl.BlockSpec((NB_C, DN), lambda i: (i, 0)),
            pl.BlockSpec((1, DN), lambda i: (0, 0)),
            pl.BlockSpec(memory_space=pltpu.SMEM),
        ],
        out_specs=pl.BlockSpec((1, NB_C), lambda i: (0, i)),
        out_shape=jax.ShapeDtypeStruct((1, N), _F32),
    )(wT, xn, nh2, wr, br)


def kernel(X, node_h, edge_h, edge_idx, mask_i, mask_ij, Wu_w, Wu_b, Wr_w, Wr_b):
    eh2 = edge_h.reshape(N, K * DE)
    mask2 = mask_ij.reshape(N, K)
    x2 = X.reshape(N, K)
    idx2 = edge_idx.reshape(N, K).astype(_I32)
    nh2 = node_h.reshape(N, DN)
    wmat = jnp.kron(jnp.eye(K, dtype=_F32), Wu_w.reshape(DE, 1))
    bu = Wu_b.reshape(1, 1)
    br = Wr_b.reshape(1, 1)

    wT, xi = _stage_a(eh2, mask2, x2, idx2, wmat, bu)
    xn = _sc_gather()(xi)
    out2 = _stage_c(wT, xn, nh2, Wr_w.reshape(1, DN), br)
    return out2.reshape(1, N, 1)
